# baseline (speedup 1.0000x reference)
import functools
import math

import jax
import jax.numpy as jnp
from jax import lax
from jax.experimental import pallas as pl
from jax.experimental.pallas import tpu as pltpu
from jax.experimental.pallas import tpu_sc as plsc

_D = 64
_LANES = 16
_NC = 2
_NS = 16
_NW = _NC * _NS
_CHUNK = 128
_BLK_CHUNKS = 40
_SCALE = math.sqrt(_D)


@functools.lru_cache(maxsize=None)
def _build(n_rows: int):
    assert n_rows % (_NW * _BLK_CHUNKS * _CHUNK) == 0
    blks_per_w = n_rows // (_NW * _BLK_CHUNKS * _CHUNK)

    mesh = plsc.VectorSubcoreMesh(
        core_axis_name="c", subcore_axis_name="s",
        num_cores=_NC, num_subcores=_NS)

    @functools.partial(
        pl.kernel,
        out_type=jax.ShapeDtypeStruct((n_rows, _D), jnp.float32),
        mesh=mesh,
        compiler_params=pltpu.CompilerParams(use_tc_tiling_on_sc=False),
        scratch_types=[
            pltpu.VMEM((_BLK_CHUNKS, _CHUNK), jnp.int32),
            pltpu.VMEM((_CHUNK, _D), jnp.float32),
            pltpu.SemaphoreType.DMA,
        ],
    )
    def sc_embed(idx_hbm, table_hbm, out_hbm, idx_v, rows_v, gsem):
        wid = lax.axis_index("s") * _NC + lax.axis_index("c")
        blk0 = wid * blks_per_w

        def blk_body(b, carry):
            pltpu.sync_copy(
                idx_hbm.at[pl.ds((blk0 + b) * _BLK_CHUNKS, _BLK_CHUNKS)],
                idx_v)

            def chunk_body(j, carry2):
                row0 = ((blk0 + b) * _BLK_CHUNKS + j) * _CHUNK
                pltpu.async_copy(table_hbm.at[idx_v.at[j]], rows_v, gsem).wait()

                def scale_body(r, carry3):
                    for k in range(_D // _LANES):
                        sl = pl.ds(k * _LANES, _LANES)
                        rows_v[r, sl] = rows_v[r, sl] * _SCALE
                    return carry3

                lax.fori_loop(0, _CHUNK, scale_body, 0, unroll=4)
                pltpu.sync_copy(rows_v, out_hbm.at[pl.ds(row0, _CHUNK)])
                return carry2

            lax.fori_loop(0, _BLK_CHUNKS, chunk_body, 0)
            return carry

        lax.fori_loop(0, blks_per_w, blk_body, 0)

    return sc_embed


def kernel(x, table):
    n_rows = x.shape[0] * x.shape[1]
    idx2d = x.reshape(n_rows // _CHUNK, _CHUNK)
    out = _build(n_rows)(idx2d, table)
    return out.reshape(x.shape[0], x.shape[1], _D)
